# baseline (speedup 1.0000x reference)
import functools

import jax
import jax.numpy as jnp
from jax import lax
from jax.experimental import pallas as pl
from jax.experimental.pallas import tpu as pltpu
from jax.experimental.pallas import tpu_sc as plsc

N, D, B = 320000, 128, 10000
B2 = 10240
NC, NS = 2, 16
ROWS_PER_W = N // NS
CHUNK = 80
NCHUNK = ROWS_PER_W // CHUNK
NPHASE = 2
PCHUNK = NCHUNK // NPHASE
B_PER_TILE = B2 // NS

_mesh = plsc.VectorSubcoreMesh(core_axis_name="c", subcore_axis_name="s")


@functools.partial(
    pl.kernel,
    out_type=(
        jax.ShapeDtypeStruct((B2, D), jnp.float32),
        jax.ShapeDtypeStruct((B2, D), jnp.float32),
    ),
    mesh=_mesh,
    scratch_types=[
        pltpu.VMEM((CHUNK, D), jnp.float32),
        pltpu.VMEM((CHUNK, D), jnp.float32),
        pltpu.VMEM((PCHUNK, CHUNK), jnp.int32),
        pltpu.VMEM_SHARED((B2, D), jnp.float32),
        pltpu.SemaphoreType.DMA,
        pltpu.SemaphoreType.DMA,
    ],
)
def _sc_aggregate(x_hbm, ids_hbm, zeros_hbm, ones_hbm, sums_hbm, counts_hbm,
                  rows_v0, rows_v1, ids_v, acc_s, sem0, sem1):
  c = lax.axis_index("c")
  s = lax.axis_index("s")
  t0 = pl.multiple_of(s * B_PER_TILE, 8)

  pltpu.sync_copy(zeros_hbm.at[pl.ds(0, CHUNK)], rows_v0)
  for k in range(B_PER_TILE // CHUNK):
    pltpu.sync_copy(rows_v0, acc_s.at[pl.ds(t0 + k * CHUNK, CHUNK)])
  plsc.subcore_barrier()

  @pl.when(c == 1)
  def _():
    pltpu.sync_copy(ones_hbm, rows_v1)

  for p in range(NPHASE):
    base = s * ROWS_PER_W + p * PCHUNK * CHUNK
    pltpu.sync_copy(ids_hbm.at[s].at[p], ids_v)

    @pl.when(c == 0)
    def _sums():
      bufs = ((rows_v0, sem0), (rows_v1, sem1))

      def _start_load(j, buf, sem):
        off = pl.multiple_of(base + jnp.minimum(j, PCHUNK - 1) * CHUNK, CHUNK)
        pltpu.async_copy(x_hbm.at[pl.ds(off, CHUNK)], buf, sem)

      def _wait_load(buf, sem):
        pltpu.make_async_copy(x_hbm.at[pl.ds(0, CHUNK)], buf, sem).wait()

      _start_load(0, rows_v0, sem0)
      _start_load(1, rows_v1, sem1)

      def body(g, carry):
        for b, (buf, sem) in enumerate(bufs):
          j = 2 * g + b
          _wait_load(buf, sem)
          pltpu.sync_copy(buf, acc_s.at[ids_v.at[j]], add=True)
          _start_load(j + 2, buf, sem)
        return carry

      lax.fori_loop(0, (PCHUNK - 1) // 2, body, 0)
      _wait_load(rows_v0, sem0)
      pltpu.sync_copy(rows_v0, acc_s.at[ids_v.at[PCHUNK - 1]], add=True)
      _wait_load(rows_v1, sem1)

    @pl.when(c == 1)
    def _counts():
      def body(j, carry):
        pltpu.sync_copy(rows_v1, acc_s.at[ids_v.at[j]], add=True)
        return carry

      lax.fori_loop(0, PCHUNK, body, 0)

  plsc.subcore_barrier()

  for k in range(B_PER_TILE // CHUNK):
    tk = pl.multiple_of(t0 + k * CHUNK, 8)
    pltpu.sync_copy(acc_s.at[pl.ds(tk, CHUNK)], rows_v0)

    @pl.when(c == 0)
    def _():
      pltpu.sync_copy(rows_v0, sums_hbm.at[pl.ds(tk, CHUNK)])

    @pl.when(c == 1)
    def _():
      pltpu.sync_copy(rows_v0, counts_hbm.at[pl.ds(tk, CHUNK)])


_BLK = 1024


def _combine_body(s_ref, c_ref, o_ref):
  cnt = c_ref[:, 0:1]
  o_ref[...] = s_ref[...] / jnp.maximum(cnt, 1.0)


_combine = pl.pallas_call(
    _combine_body,
    grid=(B2 // _BLK,),
    in_specs=[
        pl.BlockSpec((_BLK, D), lambda i: (i, 0)),
        pl.BlockSpec((_BLK, D), lambda i: (i, 0)),
    ],
    out_specs=pl.BlockSpec((_BLK, D), lambda i: (i, 0)),
    out_shape=jax.ShapeDtypeStruct((B2, D), jnp.float32),
)


@jax.jit
def kernel(X_cells, cell_to_batch, sample_idx_batch):
  del sample_idx_batch
  ids = cell_to_batch.astype(jnp.int32).reshape(NS, NPHASE, PCHUNK, CHUNK)
  zeros = jnp.zeros((CHUNK, D), jnp.float32)
  ones = jnp.ones((CHUNK, D), jnp.float32)
  sums, counts = _sc_aggregate(X_cells, ids, zeros, ones)
  return _combine(sums, counts)[:B]


# speedup vs baseline: 1.1481x; 1.1481x over previous
import functools

import jax
import jax.numpy as jnp
from jax import lax
from jax.experimental import pallas as pl
from jax.experimental.pallas import tpu as pltpu
from jax.experimental.pallas import tpu_sc as plsc

N, D, B = 320000, 128, 10000
B2 = 10240
NC, NS = 2, 16
NW = NC * NS
ROWS_PER_W = N // NW
CHUNK = 80
NCHUNK = ROWS_PER_W // CHUNK
NBUF = 3
B_PER_TILE = B2 // NS

_mesh = plsc.VectorSubcoreMesh(core_axis_name="c", subcore_axis_name="s")


def _zero_acc(zeros_hbm, rows_v, acc_s, t0):
  pltpu.sync_copy(zeros_hbm.at[pl.ds(0, CHUNK)], rows_v)
  for k in range(B_PER_TILE // CHUNK):
    pltpu.sync_copy(rows_v, acc_s.at[pl.ds(t0 + k * CHUNK, CHUNK)])


def _drain_acc(acc_s, rows_v, out_hbm_core, t0):
  for k in range(B_PER_TILE // CHUNK):
    tk = pl.multiple_of(t0 + k * CHUNK, 8)
    pltpu.sync_copy(acc_s.at[pl.ds(tk, CHUNK)], rows_v)
    pltpu.sync_copy(rows_v, out_hbm_core.at[pl.ds(tk, CHUNK)])


@functools.partial(
    pl.kernel,
    out_type=jax.ShapeDtypeStruct((NC, B2, D), jnp.float32),
    mesh=_mesh,
    scratch_types=[
        pltpu.VMEM((NBUF, CHUNK, D), jnp.float32),
        pltpu.VMEM((NCHUNK, CHUNK), jnp.int32),
        pltpu.VMEM_SHARED((B2, D), jnp.float32),
    ] + [pltpu.SemaphoreType.DMA] * NBUF,
)
def _sc_sums(x_hbm, ids_hbm, zeros_hbm, sums_hbm,
             rows_v, ids_v, acc_s, *sems):
  c = lax.axis_index("c")
  s = lax.axis_index("s")
  wid = c * NS + s
  t0 = pl.multiple_of(s * B_PER_TILE, 8)

  pltpu.sync_copy(ids_hbm.at[wid], ids_v)
  _zero_acc(zeros_hbm, rows_v.at[0], acc_s, t0)
  plsc.subcore_barrier()

  base = wid * ROWS_PER_W

  def _start_load(j, b):
    off = pl.multiple_of(base + jnp.minimum(j, NCHUNK - 1) * CHUNK, CHUNK)
    pltpu.async_copy(x_hbm.at[pl.ds(off, CHUNK)], rows_v.at[b], sems[b])

  def _wait_load(b):
    pltpu.make_async_copy(
        x_hbm.at[pl.ds(0, CHUNK)], rows_v.at[b], sems[b]).wait()

  for b in range(NBUF):
    _start_load(b, b)

  NFULL = (NCHUNK - 2) // NBUF

  def body(g, carry):
    for b in range(NBUF):
      j = NBUF * g + b
      _wait_load(b)
      pltpu.sync_copy(rows_v.at[b], acc_s.at[ids_v.at[j]], add=True)
      _start_load(j + NBUF, b)
    return carry

  lax.fori_loop(0, NFULL, body, 0)
  for b in range(NBUF):
    j = NBUF * NFULL + b
    _wait_load(b)
    if j < NCHUNK:
      pltpu.sync_copy(rows_v.at[b], acc_s.at[ids_v.at[j]], add=True)
  plsc.subcore_barrier()
  _drain_acc(acc_s, rows_v.at[0], sums_hbm.at[c], t0)


@functools.partial(
    pl.kernel,
    out_type=jax.ShapeDtypeStruct((NC, B2, D), jnp.float32),
    mesh=_mesh,
    scratch_types=[
        pltpu.VMEM((CHUNK, D), jnp.float32),
        pltpu.VMEM((CHUNK, D), jnp.float32),
        pltpu.VMEM((NCHUNK, CHUNK), jnp.int32),
        pltpu.VMEM_SHARED((B2, D), jnp.float32),
    ],
)
def _sc_counts(ids_hbm, zeros_hbm, ones_hbm, counts_hbm,
               rows_v, ones_v, ids_v, acc_s):
  c = lax.axis_index("c")
  s = lax.axis_index("s")
  wid = c * NS + s
  t0 = pl.multiple_of(s * B_PER_TILE, 8)

  pltpu.sync_copy(ids_hbm.at[wid], ids_v)
  _zero_acc(zeros_hbm, rows_v, acc_s, t0)
  pltpu.sync_copy(ones_hbm, ones_v)
  plsc.subcore_barrier()

  def body(j, carry):
    pltpu.sync_copy(ones_v, acc_s.at[ids_v.at[j]], add=True)
    return carry

  lax.fori_loop(0, NCHUNK, body, 0)
  plsc.subcore_barrier()
  _drain_acc(acc_s, rows_v, counts_hbm.at[c], t0)


_BLK = 1024


def _combine_body(s_ref, c_ref, o_ref):
  total = s_ref[0] + s_ref[1]
  cnt = c_ref[0, :, 0:1] + c_ref[1, :, 0:1]
  o_ref[...] = total / jnp.maximum(cnt, 1.0)


_combine = pl.pallas_call(
    _combine_body,
    grid=(B2 // _BLK,),
    in_specs=[
        pl.BlockSpec((NC, _BLK, D), lambda i: (0, i, 0)),
        pl.BlockSpec((NC, _BLK, D), lambda i: (0, i, 0)),
    ],
    out_specs=pl.BlockSpec((_BLK, D), lambda i: (i, 0)),
    out_shape=jax.ShapeDtypeStruct((B2, D), jnp.float32),
)


@jax.jit
def kernel(X_cells, cell_to_batch, sample_idx_batch):
  del sample_idx_batch
  ids = cell_to_batch.astype(jnp.int32).reshape(NW, NCHUNK, CHUNK)
  zeros = jnp.zeros((CHUNK, D), jnp.float32)
  ones = jnp.ones((CHUNK, D), jnp.float32)
  sums = _sc_sums(X_cells, ids, zeros)
  counts = _sc_counts(ids, zeros, ones)
  return _combine(sums, counts)[:B]


# speedup vs baseline: 1.1577x; 1.0084x over previous
import functools

import jax
import jax.numpy as jnp
from jax import lax
from jax.experimental import pallas as pl
from jax.experimental.pallas import tpu as pltpu
from jax.experimental.pallas import tpu_sc as plsc

N, D, B = 320000, 128, 10000
B2 = 10240
NC, NS = 2, 16
NW = NC * NS
ROWS_PER_W = N // NW
CHUNK = 80
NCHUNK = ROWS_PER_W // CHUNK
NBUF = 3
B_PER_TILE = B2 // NS

_mesh = plsc.VectorSubcoreMesh(core_axis_name="c", subcore_axis_name="s")


def _zero_acc(zeros_hbm, rows_v, acc_s, t0):
  pltpu.sync_copy(zeros_hbm.at[pl.ds(0, CHUNK)], rows_v)
  for k in range(B_PER_TILE // CHUNK):
    pltpu.sync_copy(rows_v, acc_s.at[pl.ds(t0 + k * CHUNK, CHUNK)])


def _drain_acc(acc_s, rows_v, out_hbm_core, t0):
  for k in range(B_PER_TILE // CHUNK):
    tk = pl.multiple_of(t0 + k * CHUNK, 8)
    pltpu.sync_copy(acc_s.at[pl.ds(tk, CHUNK)], rows_v)
    pltpu.sync_copy(rows_v, out_hbm_core.at[pl.ds(tk, CHUNK)])


@functools.partial(
    pl.kernel,
    out_type=jax.ShapeDtypeStruct((NC, B2, D), jnp.float32),
    mesh=_mesh,
    scratch_types=[
        pltpu.VMEM((NBUF, CHUNK, D), jnp.float32),
        pltpu.VMEM((NCHUNK, CHUNK), jnp.int32),
        pltpu.VMEM_SHARED((B2, D), jnp.float32),
    ] + [pltpu.SemaphoreType.DMA] * NBUF,
)
def _sc_sums(x_hbm, ids_hbm, zeros_hbm, sums_hbm,
             rows_v, ids_v, acc_s, *sems):
  c = lax.axis_index("c")
  s = lax.axis_index("s")
  wid = c * NS + s
  t0 = pl.multiple_of(s * B_PER_TILE, 8)

  pltpu.sync_copy(ids_hbm.at[wid], ids_v)
  _zero_acc(zeros_hbm, rows_v.at[0], acc_s, t0)
  plsc.subcore_barrier()

  base = wid * ROWS_PER_W

  def _start_load(j, b):
    off = pl.multiple_of(base + jnp.minimum(j, NCHUNK - 1) * CHUNK, CHUNK)
    pltpu.async_copy(x_hbm.at[pl.ds(off, CHUNK)], rows_v.at[b], sems[b])

  def _wait_load(b):
    pltpu.make_async_copy(
        x_hbm.at[pl.ds(0, CHUNK)], rows_v.at[b], sems[b]).wait()

  for b in range(NBUF):
    _start_load(b, b)

  NFULL = (NCHUNK - 2) // NBUF

  def body(g, carry):
    for b in range(NBUF):
      j = NBUF * g + b
      _wait_load(b)
      pltpu.sync_copy(rows_v.at[b], acc_s.at[ids_v.at[j]], add=True)
      _start_load(j + NBUF, b)
    return carry

  lax.fori_loop(0, NFULL, body, 0)
  for b in range(NBUF):
    j = NBUF * NFULL + b
    _wait_load(b)
    if j < NCHUNK:
      pltpu.sync_copy(rows_v.at[b], acc_s.at[ids_v.at[j]], add=True)
  plsc.subcore_barrier()
  _drain_acc(acc_s, rows_v.at[0], sums_hbm.at[c], t0)


@functools.partial(
    pl.kernel,
    out_type=jax.ShapeDtypeStruct((NC, B2, D), jnp.float32),
    mesh=_mesh,
    scratch_types=[
        pltpu.VMEM((CHUNK, D), jnp.float32),
        pltpu.VMEM((CHUNK, D), jnp.float32),
        pltpu.VMEM((NCHUNK, CHUNK), jnp.int32),
        pltpu.VMEM_SHARED((B2, D), jnp.float32),
        pltpu.SemaphoreType.DMA,
    ],
)
def _sc_counts(ids_hbm, zeros_hbm, ones_hbm, counts_hbm,
               rows_v, ones_v, ids_v, acc_s, sem):
  c = lax.axis_index("c")
  s = lax.axis_index("s")
  wid = c * NS + s
  t0 = pl.multiple_of(s * B_PER_TILE, 8)

  pltpu.sync_copy(ids_hbm.at[wid], ids_v)
  _zero_acc(zeros_hbm, rows_v, acc_s, t0)
  pltpu.sync_copy(ones_hbm, ones_v)
  plsc.subcore_barrier()

  def _start_scatter(j):
    pltpu.async_copy(ones_v, acc_s.at[ids_v.at[j]], sem, add=True)

  def _wait_scatter():
    pltpu.make_async_copy(ones_v, acc_s.at[ids_v.at[0]], sem).wait()

  _start_scatter(0)
  _start_scatter(1)

  def body(j, carry):
    _start_scatter(j + 2)
    _wait_scatter()
    return carry

  lax.fori_loop(0, NCHUNK - 2, body, 0)
  _wait_scatter()
  _wait_scatter()
  plsc.subcore_barrier()
  _drain_acc(acc_s, rows_v, counts_hbm.at[c], t0)


_BLK = 1024


def _combine_body(s_ref, c_ref, o_ref):
  total = s_ref[0] + s_ref[1]
  cnt = c_ref[0, :, 0:1] + c_ref[1, :, 0:1]
  o_ref[...] = total / jnp.maximum(cnt, 1.0)


_combine = pl.pallas_call(
    _combine_body,
    grid=(B2 // _BLK,),
    in_specs=[
        pl.BlockSpec((NC, _BLK, D), lambda i: (0, i, 0)),
        pl.BlockSpec((NC, _BLK, D), lambda i: (0, i, 0)),
    ],
    out_specs=pl.BlockSpec((_BLK, D), lambda i: (i, 0)),
    out_shape=jax.ShapeDtypeStruct((B2, D), jnp.float32),
)


@jax.jit
def kernel(X_cells, cell_to_batch, sample_idx_batch):
  del sample_idx_batch
  ids = cell_to_batch.astype(jnp.int32).reshape(NW, NCHUNK, CHUNK)
  zeros = jnp.zeros((CHUNK, D), jnp.float32)
  ones = jnp.ones((CHUNK, D), jnp.float32)
  sums = _sc_sums(X_cells, ids, zeros)
  counts = _sc_counts(ids, zeros, ones)
  return _combine(sums, counts)[:B]
